# baseline (speedup 1.0000x reference)
import functools

import jax
import jax.numpy as jnp
from jax import lax
from jax.experimental import pallas as pl
from jax.experimental.pallas import tpu as pltpu
from jax.experimental.pallas import tpu_sc as plsc

_NUM_SPECIAL = 16
_BLOCK_S = 512


def _bulk_body(w_ref, o_ref):
    nb = o_ref.shape[0]
    bs = o_ref.shape[1]
    rows = jnp.broadcast_to(w_ref[0], (bs, o_ref.shape[2]))
    for b in range(nb):
        o_ref[b] = rows


def _stage_body(w_hbm, idx_hbm, rows_hbm, iv_hbm, idx_v, ivv, w16_v, sem,
                isem, B, S, H):
    info = plsc.get_sparse_core_info()
    nc = info.num_cores
    wid = lax.axis_index("s") * nc + lax.axis_index("c")

    @pl.when(wid < B)
    def _():
        ones = jnp.ones((_NUM_SPECIAL,), jnp.int32)
        hw = pltpu.async_copy(w_hbm.at[ones], w16_v, sem)
        hi = pltpu.async_copy(idx_hbm, idx_v, isem)
        hi.wait()
        ivv[...] = idx_v[...] + wid * S
        hw.wait()
        ho = pltpu.async_copy(w16_v, rows_hbm.at[pl.ds(wid * _NUM_SPECIAL,
                                                       _NUM_SPECIAL)], sem)
        hv = pltpu.async_copy(ivv, iv_hbm.at[wid], isem)
        ho.wait()
        hv.wait()


def _apply_body(iv_ref, rows_ref, bulk_ref, o_ref, sem):
    handles = []
    for b in range(iv_ref.shape[0]):
        for j in range(iv_ref.shape[1]):
            row = iv_ref[b, j]
            handles.append(pltpu.make_async_copy(
                rows_ref.at[pl.ds(b * iv_ref.shape[1] + j, 1)],
                o_ref.at[pl.ds(row, 1)], sem))
    for h in handles:
        h.start()
    for h in handles:
        h.wait()


def kernel(x, special_tokens_indices, W):
    B, S, H = x.shape
    idx = special_tokens_indices.astype(jnp.int32)

    stage = functools.partial(
        pl.kernel,
        mesh=plsc.VectorSubcoreMesh(core_axis_name="c", subcore_axis_name="s"),
        out_type=(
            jax.ShapeDtypeStruct((B * _NUM_SPECIAL, H), jnp.float32),
            jax.ShapeDtypeStruct((B, _NUM_SPECIAL), jnp.int32),
        ),
        scratch_types=[
            pltpu.VMEM((_NUM_SPECIAL,), jnp.int32),
            pltpu.VMEM((_NUM_SPECIAL,), jnp.int32),
            pltpu.VMEM((_NUM_SPECIAL, H), jnp.float32),
            pltpu.SemaphoreType.DMA,
            pltpu.SemaphoreType.DMA,
        ],
    )(functools.partial(_stage_body, B=B, S=S, H=H))
    rows64, iv = stage(W, idx)

    bulk = pl.pallas_call(
        _bulk_body,
        grid=(S // _BLOCK_S,),
        in_specs=[pl.BlockSpec((2, H), lambda s: (0, 0))],
        out_specs=pl.BlockSpec((B, _BLOCK_S, H), lambda s: (0, s, 0)),
        out_shape=jax.ShapeDtypeStruct((B, S, H), jnp.float32),
        compiler_params=pltpu.CompilerParams(
            dimension_semantics=("arbitrary",),
        ),
    )(W)

    out = pl.pallas_call(
        _apply_body,
        in_specs=[
            pl.BlockSpec(memory_space=pltpu.SMEM),
            pl.BlockSpec(memory_space=pl.ANY),
            pl.BlockSpec(memory_space=pl.ANY),
        ],
        out_specs=pl.BlockSpec(memory_space=pl.ANY),
        out_shape=jax.ShapeDtypeStruct((B * S, H), jnp.float32),
        scratch_shapes=[pltpu.SemaphoreType.DMA],
        input_output_aliases={2: 0},
    )(iv, rows64, bulk.reshape(B * S, H))
    return out.reshape(B, S, H)


# speedup vs baseline: 1.1878x; 1.1878x over previous
import functools

import jax
import jax.numpy as jnp
from jax import lax
from jax.experimental import pallas as pl
from jax.experimental.pallas import tpu as pltpu
from jax.experimental.pallas import tpu_sc as plsc

_NUM_SPECIAL = 16
_BLOCK_S = 512


def _bulk_body(w_ref, o_ref):
    nb = o_ref.shape[0]
    bs = o_ref.shape[1]
    rows = jnp.broadcast_to(w_ref[0], (bs, o_ref.shape[2]))
    for b in range(nb):
        o_ref[b] = rows


def _patch_body(w_hbm, idx_hbm, out_ref, idx_v, w16_v, sem, isem, B, S, H):
    info = plsc.get_sparse_core_info()
    nc = info.num_cores
    wid = lax.axis_index("s") * nc + lax.axis_index("c")

    @pl.when(wid < B)
    def _():
        ones = jnp.ones((_NUM_SPECIAL,), jnp.int32)
        hw = pltpu.async_copy(w_hbm.at[ones], w16_v, sem)
        hi = pltpu.async_copy(idx_hbm, idx_v, isem)
        hi.wait()
        hw.wait()
        iv = idx_v[...] + wid * S
        pltpu.async_copy(w16_v, out_ref.at[iv], sem).wait()


def kernel(x, special_tokens_indices, W):
    B, S, H = x.shape
    idx = special_tokens_indices.astype(jnp.int32)

    bulk = pl.pallas_call(
        _bulk_body,
        grid=(S // _BLOCK_S,),
        in_specs=[pl.BlockSpec((2, H), lambda s: (0, 0))],
        out_specs=pl.BlockSpec((B, _BLOCK_S, H), lambda s: (0, s, 0)),
        out_shape=jax.ShapeDtypeStruct((B, S, H), jnp.float32),
        compiler_params=pltpu.CompilerParams(
            dimension_semantics=("arbitrary",),
        ),
    )(W)

    out_ref = jax.new_ref(bulk.reshape(B * S, H))
    patch = functools.partial(
        pl.kernel,
        mesh=plsc.VectorSubcoreMesh(core_axis_name="c", subcore_axis_name="s", num_cores=1),
        scratch_types=[
            pltpu.VMEM((_NUM_SPECIAL,), jnp.int32),
            pltpu.VMEM((_NUM_SPECIAL, H), jnp.float32),
            pltpu.SemaphoreType.DMA,
            pltpu.SemaphoreType.DMA,
        ],
    )(functools.partial(_patch_body, B=B, S=S, H=H))
    patch(W, idx, out_ref)
    return out_ref[...].reshape(B, S, H)
